# baseline (speedup 1.0000x reference)
import functools

import jax
import jax.numpy as jnp
from jax import lax
from jax.experimental import pallas as pl
from jax.experimental.pallas import tpu as pltpu
from jax.experimental.pallas import tpu_sc as plsc

NUM_FIELDS = 26
HALF_FIELDS = 13
EMBED_DIM = 416
EMBED_PAD = 512
BLK = 16
BATCH = 4096
FIELD_SIZE = 2000
N_ROWS = NUM_FIELDS * FIELD_SIZE
SPLIT_ROW = HALF_FIELDS * FIELD_SIZE

NC, NS = 2, 16
NW = NC * NS
CB = BATCH // NW
G = 4
ROWS = G * HALF_FIELDS
IDXC = 56
NCHUNK = CB // G

PCOLS = 640
E0_OFF = EMBED_DIM

TBLK = 512
B_START = 25600
A_ROWS = 26112
B_ROWS = 26624

_MESH = plsc.VectorSubcoreMesh(core_axis_name="c", subcore_axis_name="s")


def _transpose_body(v_ref, o_ref):
    o_ref[:, :EMBED_DIM] = jnp.transpose(v_ref[...])
    o_ref[:, EMBED_DIM:] = jnp.zeros((TBLK, EMBED_PAD - EMBED_DIM), jnp.float32)


def _make_transpose(n_out_rows, col_block_off):
    return pl.pallas_call(
        _transpose_body,
        grid=(n_out_rows // TBLK,),
        in_specs=[pl.BlockSpec((EMBED_DIM, TBLK), lambda i: (0, i + col_block_off))],
        out_specs=pl.BlockSpec((TBLK, EMBED_PAD), lambda i: (i, 0)),
        out_shape=jax.ShapeDtypeStruct((n_out_rows, EMBED_PAD), jnp.float32),
    )


_transpose_a = _make_transpose(A_ROWS, 0)
_transpose_b = _make_transpose(B_ROWS, B_START // TBLK)

_SC_PARAMS = pltpu.CompilerParams(
    needs_layout_passes=False, use_tc_tiling_on_sc=True
)


def _gather_ring(idx_v, w_hbm, rows_v, sems, compute_chunk):
    def fire(c, b):
        pltpu.async_copy(w_hbm.at[idx_v.at[c]], rows_v.at[b], sems[b])

    def wait(c, b):
        pltpu.make_async_copy(w_hbm.at[idx_v.at[c]], rows_v.at[b], sems[b]).wait()

    fire(0, 0)
    fire(1, 1)

    def outer(g2, carry):
        for b in range(2):
            c = 2 * g2 + b
            wait(c, b)
            compute_chunk(c, b)
            fire(c + 2, b)
        return carry

    lax.fori_loop(0, NCHUNK // 2 - 1, outer, 0)

    for b in range(2):
        c = NCHUNK - 2 + b
        wait(c, b)
        compute_chunk(c, b)


@functools.partial(
    pl.kernel,
    out_type=jax.ShapeDtypeStruct((BATCH, PCOLS), jnp.float32),
    mesh=_MESH,
    scratch_types=[
        pltpu.VMEM((NCHUNK, IDXC), jnp.int32),
        pltpu.VMEM((2 * G, PCOLS), jnp.float32),
        pltpu.VMEM((2, IDXC, EMBED_PAD), jnp.float32),
        pltpu.SemaphoreType.DMA,
        pltpu.SemaphoreType.DMA,
    ],
    compiler_params=_SC_PARAMS,
)
def _fwlm_sc1(idx_hbm, wa_hbm, p_hbm, idx_v, pout_v, rows_v, sem0, sem1):
    wid = lax.axis_index("s") * NC + lax.axis_index("c")

    pltpu.sync_copy(idx_hbm.at[wid], idx_v)

    def compute_chunk(c, b):
        def elem_body(e, carry):
            base = e * HALF_FIELDS
            prow = b * G + e
            for j in range(NUM_FIELDS):
                col = pl.ds(BLK * j, BLK)
                s = rows_v[b, base, col]
                if j == 0:
                    pout_v[prow, pl.ds(E0_OFF, BLK)] = s
                for f in range(1, HALF_FIELDS):
                    v = rows_v[b, base + f, col]
                    if f == j:
                        pout_v[prow, pl.ds(E0_OFF + BLK * j, BLK)] = v
                    s = s + v
                pout_v[prow, pl.ds(BLK * j, BLK)] = s
            return carry

        lax.fori_loop(0, G, elem_body, 0)
        if b == 1:
            pltpu.sync_copy(
                pout_v, p_hbm.at[pl.ds(wid * CB + (c // 2) * 2 * G, 2 * G)]
            )

    _gather_ring(idx_v, wa_hbm, rows_v, (sem0, sem1), compute_chunk)


@functools.partial(
    pl.kernel,
    out_type=jax.ShapeDtypeStruct((BATCH,), jnp.float32),
    mesh=_MESH,
    scratch_types=[
        pltpu.VMEM((NCHUNK, IDXC), jnp.int32),
        pltpu.VMEM((2 * G, PCOLS), jnp.float32),
        pltpu.VMEM((2, IDXC, EMBED_PAD), jnp.float32),
        pltpu.VMEM((CB,), jnp.float32),
        pltpu.SemaphoreType.DMA,
        pltpu.SemaphoreType.DMA,
    ],
    compiler_params=_SC_PARAMS,
)
def _fwlm_sc2(idx_hbm, wb_hbm, p_hbm, out_hbm, idx_v, pin_v, rows_v, out_v, sem0, sem1):
    wid = lax.axis_index("s") * NC + lax.axis_index("c")
    lanes = lax.iota(jnp.int32, BLK)

    for i in range(CB // BLK):
        out_v[pl.ds(i * BLK, BLK)] = jnp.zeros((BLK,), jnp.float32)

    pltpu.sync_copy(idx_hbm.at[wid], idx_v)

    def compute_chunk(c, b):
        if b == 0:
            pltpu.sync_copy(
                p_hbm.at[pl.ds(wid * CB + (c // 2) * 2 * G, 2 * G)], pin_v
            )

        def elem_body(e, carry):
            base = e * HALF_FIELDS
            prow = b * G + e
            acc = jnp.zeros((BLK,), jnp.float32)
            for j in range(NUM_FIELDS):
                col = pl.ds(BLK * j, BLK)
                s = rows_v[b, base, col]
                ej = s
                for f in range(1, HALF_FIELDS):
                    v = rows_v[b, base + f, col]
                    if f == j - HALF_FIELDS:
                        ej = v
                    s = s + v
                s = s + pin_v[prow, col]
                if j < HALF_FIELDS:
                    ej = pin_v[prow, pl.ds(E0_OFF + BLK * j, BLK)]
                acc = acc + ej * (s - ej)
            s_val = jnp.sum(acc)
            pos = c * G + e
            grp = (pos // BLK) * BLK
            cur = out_v[pl.ds(grp, BLK)]
            out_v[pl.ds(grp, BLK)] = cur + jnp.where(
                lanes == pos % BLK, s_val, 0.0
            )
            return carry

        lax.fori_loop(0, G, elem_body, 0)

    _gather_ring(idx_v, wb_hbm, rows_v, (sem0, sem1), compute_chunk)

    pltpu.sync_copy(out_v, out_hbm.at[pl.ds(wid * CB, CB)])


def kernel(x, W, bias):
    xi = x.astype(jnp.int32)
    offs = jnp.arange(NUM_FIELDS, dtype=jnp.int32) * FIELD_SIZE
    idx_a = (xi[:, :HALF_FIELDS] + offs[:HALF_FIELDS]).reshape(NW, NCHUNK, ROWS)
    idx_b = (xi[:, HALF_FIELDS:] + offs[HALF_FIELDS:] - B_START).reshape(
        NW, NCHUNK, ROWS
    )
    pad = ((0, 0), (0, 0), (0, IDXC - ROWS))
    idx_a = jnp.pad(idx_a, pad)
    idx_b = jnp.pad(idx_b, pad)
    v = W.T
    wa = _transpose_a(v)
    partials = _fwlm_sc1(idx_a, wa)
    wb = _transpose_b(v)
    out = _fwlm_sc2(idx_b, wb, partials)
    return out + bias[0]


# speedup vs baseline: 2.3506x; 2.3506x over previous
import functools

import jax
import jax.numpy as jnp
from jax import lax
from jax.experimental import pallas as pl
from jax.experimental.pallas import tpu as pltpu
from jax.experimental.pallas import tpu_sc as plsc

NUM_FIELDS = 26
EMBED_DIM = 416
EMBED_PAD = 512
BLK = 16
BATCH = 4096
FIELD_SIZE = 2000
N_ROWS = NUM_FIELDS * FIELD_SIZE

NC, NS = 2, 16
NW = NC * NS
CB = BATCH // NW
G = 4
ROWS = G * NUM_FIELDS
NCHUNK = CB // G

TBLK = 512
TGRID = -(-N_ROWS // TBLK)

_MESH = plsc.VectorSubcoreMesh(core_axis_name="c", subcore_axis_name="s")


def _transpose_body(v_ref, o_ref):
    o_ref[:, :EMBED_DIM] = jnp.transpose(v_ref[...])
    o_ref[:, EMBED_DIM:] = jnp.zeros((TBLK, EMBED_PAD - EMBED_DIM), jnp.float32)


_transpose_tc = pl.pallas_call(
    _transpose_body,
    grid=(TGRID,),
    in_specs=[pl.BlockSpec((EMBED_DIM, TBLK), lambda i: (0, i))],
    out_specs=pl.BlockSpec((TBLK, EMBED_PAD), lambda i: (i, 0)),
    out_shape=jax.ShapeDtypeStruct((N_ROWS, EMBED_PAD), jnp.float32),
)


@functools.partial(
    pl.kernel,
    out_type=jax.ShapeDtypeStruct((BATCH,), jnp.float32),
    mesh=_MESH,
    scratch_types=[
        pltpu.VMEM((NCHUNK, ROWS), jnp.int32),
        pltpu.VMEM((2, ROWS, EMBED_PAD), jnp.float32),
        pltpu.VMEM((CB,), jnp.float32),
        pltpu.SemaphoreType.DMA,
        pltpu.SemaphoreType.DMA,
    ],
    compiler_params=pltpu.CompilerParams(
        needs_layout_passes=False, use_tc_tiling_on_sc=True
    ),
)
def _fwlm_sc(idx_hbm, w_hbm, out_hbm, idx_v, rows_v, out_v, sem0, sem1):
    wid = lax.axis_index("s") * NC + lax.axis_index("c")
    sems = (sem0, sem1)
    lanes = lax.iota(jnp.int32, BLK)

    for i in range(CB // BLK):
        out_v[pl.ds(i * BLK, BLK)] = jnp.zeros((BLK,), jnp.float32)

    pltpu.sync_copy(idx_hbm.at[wid], idx_v)

    def fire(c, b):
        pltpu.async_copy(w_hbm.at[idx_v.at[c]], rows_v.at[b], sems[b])

    def wait(c, b):
        pltpu.make_async_copy(w_hbm.at[idx_v.at[c]], rows_v.at[b], sems[b]).wait()

    def compute_chunk(c, b):
        def elem_body(e, carry):
            base = e * NUM_FIELDS
            acc = jnp.zeros((BLK,), jnp.float32)
            for j in range(NUM_FIELDS):
                col = pl.ds(BLK * j, BLK)
                s = rows_v[b, base, col]
                ej = s
                for f in range(1, NUM_FIELDS):
                    v = rows_v[b, base + f, col]
                    if f == j:
                        ej = v
                    s = s + v
                acc = acc + ej * (s - ej)
            s_val = jnp.sum(acc)
            pos = c * G + e
            grp = (pos // BLK) * BLK
            cur = out_v[pl.ds(grp, BLK)]
            out_v[pl.ds(grp, BLK)] = cur + jnp.where(
                lanes == pos % BLK, s_val, 0.0
            )
            return carry

        lax.fori_loop(0, G, elem_body, 0)

    fire(0, 0)
    fire(1, 1)

    def outer(g2, carry):
        for b in range(2):
            c = 2 * g2 + b
            wait(c, b)
            compute_chunk(c, b)
            fire(c + 2, b)
        return carry

    lax.fori_loop(0, NCHUNK // 2 - 1, outer, 0)

    for b in range(2):
        c = NCHUNK - 2 + b
        wait(c, b)
        compute_chunk(c, b)

    pltpu.sync_copy(out_v, out_hbm.at[pl.ds(wid * CB, CB)])


def kernel(x, W, bias):
    offs = (jnp.arange(NUM_FIELDS, dtype=jnp.int32) * FIELD_SIZE)[None, :]
    idx = (x.astype(jnp.int32) + offs).reshape(NW, NCHUNK, ROWS)
    w_pad = _transpose_tc(W.T)
    out = _fwlm_sc(idx, w_pad)
    return out + bias[0]


# speedup vs baseline: 2.6758x; 1.1383x over previous
import functools

import jax
import jax.numpy as jnp
from jax import lax
from jax.experimental import pallas as pl
from jax.experimental.pallas import tpu as pltpu
from jax.experimental.pallas import tpu_sc as plsc

NUM_FIELDS = 26
EMBED_DIM = 416
EMBED_PAD = 512
BLK = 16
BATCH = 4096
FIELD_SIZE = 2000
N_ROWS = NUM_FIELDS * FIELD_SIZE

NC, NS = 2, 16
NW = NC * NS
CB = BATCH // NW
G = 4
ROWS = G * NUM_FIELDS
NCHUNK = CB // G

TBLK = 1024
TGRID = -(-N_ROWS // TBLK)

_MESH = plsc.VectorSubcoreMesh(core_axis_name="c", subcore_axis_name="s")


def _transpose_body(v_ref, o_ref):
    o_ref[:, :EMBED_DIM] = jnp.transpose(v_ref[...])
    o_ref[:, EMBED_DIM:] = jnp.zeros((TBLK, EMBED_PAD - EMBED_DIM), jnp.float32)


_transpose_tc = pl.pallas_call(
    _transpose_body,
    grid=(TGRID,),
    in_specs=[pl.BlockSpec((EMBED_DIM, TBLK), lambda i: (0, i))],
    out_specs=pl.BlockSpec((TBLK, EMBED_PAD), lambda i: (i, 0)),
    out_shape=jax.ShapeDtypeStruct((N_ROWS, EMBED_PAD), jnp.float32),
)


@functools.partial(
    pl.kernel,
    out_type=jax.ShapeDtypeStruct((BATCH,), jnp.float32),
    mesh=_MESH,
    scratch_types=[
        pltpu.VMEM((NCHUNK, ROWS), jnp.int32),
        pltpu.VMEM((2, ROWS, EMBED_PAD), jnp.float32),
        pltpu.VMEM((CB,), jnp.float32),
        pltpu.SemaphoreType.DMA,
        pltpu.SemaphoreType.DMA,
    ],
    compiler_params=pltpu.CompilerParams(
        needs_layout_passes=False, use_tc_tiling_on_sc=True
    ),
)
def _fwlm_sc(idx_hbm, w_hbm, out_hbm, idx_v, rows_v, out_v, sem0, sem1):
    wid = lax.axis_index("s") * NC + lax.axis_index("c")
    sems = (sem0, sem1)
    lanes = lax.iota(jnp.int32, BLK)

    for i in range(CB // BLK):
        out_v[pl.ds(i * BLK, BLK)] = jnp.zeros((BLK,), jnp.float32)

    pltpu.sync_copy(idx_hbm.at[wid], idx_v)

    def fire(c, b):
        pltpu.async_copy(w_hbm.at[idx_v.at[c]], rows_v.at[b], sems[b])

    def wait(c, b):
        pltpu.make_async_copy(w_hbm.at[idx_v.at[c]], rows_v.at[b], sems[b]).wait()

    def compute_chunk(c, b):
        def elem_body(e, carry):
            base = e * NUM_FIELDS
            acc = jnp.zeros((BLK,), jnp.float32)
            for j in range(NUM_FIELDS):
                col = pl.ds(BLK * j, BLK)
                s = rows_v[b, base, col]
                ej = s
                for f in range(1, NUM_FIELDS):
                    v = rows_v[b, base + f, col]
                    if f == j:
                        ej = v
                    s = s + v
                acc = acc + ej * (s - ej)
            s_val = jnp.sum(acc)
            pos = c * G + e
            grp = (pos // BLK) * BLK
            cur = out_v[pl.ds(grp, BLK)]
            out_v[pl.ds(grp, BLK)] = cur + jnp.where(
                lanes == pos % BLK, s_val, 0.0
            )
            return carry

        lax.fori_loop(0, G, elem_body, 0)

    fire(0, 0)
    fire(1, 1)

    def outer(g2, carry):
        for b in range(2):
            c = 2 * g2 + b
            wait(c, b)
            compute_chunk(c, b)
            fire(c + 2, b)
        return carry

    lax.fori_loop(0, NCHUNK // 2 - 1, outer, 0)

    for b in range(2):
        c = NCHUNK - 2 + b
        wait(c, b)
        compute_chunk(c, b)

    pltpu.sync_copy(out_v, out_hbm.at[pl.ds(wid * CB, CB)])


def kernel(x, W, bias):
    offs = (jnp.arange(NUM_FIELDS, dtype=jnp.int32) * FIELD_SIZE)[None, :]
    idx = (x.astype(jnp.int32) + offs).reshape(NW, NCHUNK, ROWS)
    w_pad = _transpose_tc(W.T)
    out = _fwlm_sc(idx, w_pad)
    return out + bias[0]


# speedup vs baseline: 2.8438x; 1.0628x over previous
import functools

import jax
import jax.numpy as jnp
from jax import lax
from jax.experimental import pallas as pl
from jax.experimental.pallas import tpu as pltpu
from jax.experimental.pallas import tpu_sc as plsc

NUM_FIELDS = 26
EMBED_DIM = 416
EMBED_PAD = 512
BLK = 16
BATCH = 4096
FIELD_SIZE = 2000
N_ROWS = NUM_FIELDS * FIELD_SIZE

NC, NS = 2, 16
NW = NC * NS
CB = BATCH // NW
G = 4
ROWS = G * NUM_FIELDS
NCHUNK = CB // G

TBLK = 2048
TGRID = -(-N_ROWS // TBLK)

_MESH = plsc.VectorSubcoreMesh(core_axis_name="c", subcore_axis_name="s")


def _transpose_body(v_ref, o_ref):
    o_ref[:, :EMBED_DIM] = jnp.transpose(v_ref[...])
    o_ref[:, EMBED_DIM:] = jnp.zeros((TBLK, EMBED_PAD - EMBED_DIM), jnp.float32)


_transpose_tc = pl.pallas_call(
    _transpose_body,
    grid=(TGRID,),
    in_specs=[pl.BlockSpec((EMBED_DIM, TBLK), lambda i: (0, i))],
    out_specs=pl.BlockSpec((TBLK, EMBED_PAD), lambda i: (i, 0)),
    out_shape=jax.ShapeDtypeStruct((N_ROWS, EMBED_PAD), jnp.float32),
)


@functools.partial(
    pl.kernel,
    out_type=jax.ShapeDtypeStruct((BATCH,), jnp.float32),
    mesh=_MESH,
    scratch_types=[
        pltpu.VMEM((NCHUNK, ROWS), jnp.int32),
        pltpu.VMEM((2, ROWS, EMBED_PAD), jnp.float32),
        pltpu.VMEM((CB,), jnp.float32),
        pltpu.SemaphoreType.DMA,
        pltpu.SemaphoreType.DMA,
    ],
    compiler_params=pltpu.CompilerParams(
        needs_layout_passes=False, use_tc_tiling_on_sc=True
    ),
)
def _fwlm_sc(idx_hbm, w_hbm, out_hbm, idx_v, rows_v, out_v, sem0, sem1):
    wid = lax.axis_index("s") * NC + lax.axis_index("c")
    sems = (sem0, sem1)
    lanes = lax.iota(jnp.int32, BLK)

    for i in range(CB // BLK):
        out_v[pl.ds(i * BLK, BLK)] = jnp.zeros((BLK,), jnp.float32)

    pltpu.sync_copy(idx_hbm.at[wid], idx_v)

    def fire(c, b):
        pltpu.async_copy(w_hbm.at[idx_v.at[c]], rows_v.at[b], sems[b])

    def wait(c, b):
        pltpu.make_async_copy(w_hbm.at[idx_v.at[c]], rows_v.at[b], sems[b]).wait()

    def compute_chunk(c, b):
        def elem_body(e, carry):
            base = e * NUM_FIELDS
            acc = jnp.zeros((BLK,), jnp.float32)
            for j in range(NUM_FIELDS):
                col = pl.ds(BLK * j, BLK)
                s = rows_v[b, base, col]
                ej = s
                for f in range(1, NUM_FIELDS):
                    v = rows_v[b, base + f, col]
                    if f == j:
                        ej = v
                    s = s + v
                acc = acc + ej * (s - ej)
            s_val = jnp.sum(acc)
            pos = c * G + e
            grp = (pos // BLK) * BLK
            cur = out_v[pl.ds(grp, BLK)]
            out_v[pl.ds(grp, BLK)] = cur + jnp.where(
                lanes == pos % BLK, s_val, 0.0
            )
            return carry

        lax.fori_loop(0, G, elem_body, 0)

    fire(0, 0)
    fire(1, 1)

    def outer(g2, carry):
        for b in range(2):
            c = 2 * g2 + b
            wait(c, b)
            compute_chunk(c, b)
            fire(c + 2, b)
        return carry

    lax.fori_loop(0, NCHUNK // 2 - 1, outer, 0)

    for b in range(2):
        c = NCHUNK - 2 + b
        wait(c, b)
        compute_chunk(c, b)

    pltpu.sync_copy(out_v, out_hbm.at[pl.ds(wid * CB, CB)])


def kernel(x, W, bias):
    offs = (jnp.arange(NUM_FIELDS, dtype=jnp.int32) * FIELD_SIZE)[None, :]
    idx = (x.astype(jnp.int32) + offs).reshape(NW, NCHUNK, ROWS)
    w_pad = _transpose_tc(W.T)
    out = _fwlm_sc(idx, w_pad)
    return out + bias[0]


# speedup vs baseline: 2.8923x; 1.0171x over previous
import functools

import jax
import jax.numpy as jnp
from jax import lax
from jax.experimental import pallas as pl
from jax.experimental.pallas import tpu as pltpu
from jax.experimental.pallas import tpu_sc as plsc

NUM_FIELDS = 26
EMBED_DIM = 416
EMBED_PAD = 512
BLK = 16
BATCH = 4096
FIELD_SIZE = 2000
N_ROWS = NUM_FIELDS * FIELD_SIZE

NC, NS = 2, 16
NW = NC * NS
CB = BATCH // NW
G = 4
ROWS = G * NUM_FIELDS
NCHUNK = CB // G

TBLK = 4096
TGRID = -(-N_ROWS // TBLK)

_MESH = plsc.VectorSubcoreMesh(core_axis_name="c", subcore_axis_name="s")


def _transpose_body(v_ref, o_ref):
    o_ref[:, :EMBED_DIM] = jnp.transpose(v_ref[...])
    o_ref[:, EMBED_DIM:] = jnp.zeros((TBLK, EMBED_PAD - EMBED_DIM), jnp.float32)


_transpose_tc = pl.pallas_call(
    _transpose_body,
    grid=(TGRID,),
    in_specs=[pl.BlockSpec((EMBED_DIM, TBLK), lambda i: (0, i))],
    out_specs=pl.BlockSpec((TBLK, EMBED_PAD), lambda i: (i, 0)),
    out_shape=jax.ShapeDtypeStruct((N_ROWS, EMBED_PAD), jnp.float32),
)


@functools.partial(
    pl.kernel,
    out_type=jax.ShapeDtypeStruct((BATCH,), jnp.float32),
    mesh=_MESH,
    scratch_types=[
        pltpu.VMEM((NCHUNK, ROWS), jnp.int32),
        pltpu.VMEM((2, ROWS, EMBED_PAD), jnp.float32),
        pltpu.VMEM((CB,), jnp.float32),
        pltpu.SemaphoreType.DMA,
        pltpu.SemaphoreType.DMA,
    ],
    compiler_params=pltpu.CompilerParams(
        needs_layout_passes=False, use_tc_tiling_on_sc=True
    ),
)
def _fwlm_sc(idx_hbm, w_hbm, out_hbm, idx_v, rows_v, out_v, sem0, sem1):
    wid = lax.axis_index("s") * NC + lax.axis_index("c")
    sems = (sem0, sem1)
    lanes = lax.iota(jnp.int32, BLK)

    for i in range(CB // BLK):
        out_v[pl.ds(i * BLK, BLK)] = jnp.zeros((BLK,), jnp.float32)

    pltpu.sync_copy(idx_hbm.at[wid], idx_v)

    def fire(c, b):
        pltpu.async_copy(w_hbm.at[idx_v.at[c]], rows_v.at[b], sems[b])

    def wait(c, b):
        pltpu.make_async_copy(w_hbm.at[idx_v.at[c]], rows_v.at[b], sems[b]).wait()

    def compute_chunk(c, b):
        def elem_body(e, carry):
            base = e * NUM_FIELDS
            acc = jnp.zeros((BLK,), jnp.float32)
            for j in range(NUM_FIELDS):
                col = pl.ds(BLK * j, BLK)
                s = rows_v[b, base, col]
                ej = s
                for f in range(1, NUM_FIELDS):
                    v = rows_v[b, base + f, col]
                    if f == j:
                        ej = v
                    s = s + v
                acc = acc + ej * (s - ej)
            s_val = jnp.sum(acc)
            pos = c * G + e
            grp = (pos // BLK) * BLK
            cur = out_v[pl.ds(grp, BLK)]
            out_v[pl.ds(grp, BLK)] = cur + jnp.where(
                lanes == pos % BLK, s_val, 0.0
            )
            return carry

        lax.fori_loop(0, G, elem_body, 0)

    fire(0, 0)
    fire(1, 1)

    def outer(g2, carry):
        for b in range(2):
            c = 2 * g2 + b
            wait(c, b)
            compute_chunk(c, b)
            fire(c + 2, b)
        return carry

    lax.fori_loop(0, NCHUNK // 2 - 1, outer, 0)

    for b in range(2):
        c = NCHUNK - 2 + b
        wait(c, b)
        compute_chunk(c, b)

    pltpu.sync_copy(out_v, out_hbm.at[pl.ds(wid * CB, CB)])


def kernel(x, W, bias):
    offs = (jnp.arange(NUM_FIELDS, dtype=jnp.int32) * FIELD_SIZE)[None, :]
    idx = (x.astype(jnp.int32) + offs).reshape(NW, NCHUNK, ROWS)
    w_pad = _transpose_tc(W.T)
    out = _fwlm_sc(idx, w_pad)
    return out + bias[0]
